# baseline (speedup 1.0000x reference)
import functools

import jax
import jax.numpy as jnp
from jax import lax
from jax.experimental import pallas as pl
from jax.experimental.pallas import tpu as pltpu
from jax.experimental.pallas import tpu_sc as plsc

_DIM = 32
_LAMBDA = 0.01
_CHUNK = 64
_EB = 4096


def _pack_tables(ereT, eimT, n_ent):
    n_blocks = (n_ent + _EB - 1) // _EB
    n_sub = _EB // 512

    def pack16(x_ref, sl):
        lo = lax.bitcast_convert_type(
            x_ref[0:16, sl].astype(jnp.bfloat16), jnp.uint16).astype(jnp.int32)
        hi = lax.bitcast_convert_type(
            x_ref[16:32, sl].astype(jnp.bfloat16), jnp.uint16).astype(jnp.int32)
        return lo | (hi << 16)

    def body(a_ref, b_ref, o_ref):
        for k in range(n_sub):
            pieces = []
            for kk in range(4):
                sl = pl.ds(512 * k + 128 * kk, 128)
                pieces.append(pack16(a_ref, sl))
                pieces.append(pack16(b_ref, sl))
            stacked = jnp.concatenate(pieces, axis=0)
            o_ref[pl.ds(128 * k, 128), :] = stacked.T

    return pl.pallas_call(
        body,
        grid=(n_blocks,),
        in_specs=[
            pl.BlockSpec((_DIM, _EB), lambda g: (0, g)),
            pl.BlockSpec((_DIM, _EB), lambda g: (0, g)),
        ],
        out_specs=pl.BlockSpec((_EB // 4, 128), lambda g: (g, 0)),
        out_shape=jax.ShapeDtypeStruct((n_blocks * (_EB // 4), 128), jnp.int32),
    )(ereT, eimT)


def _pack_rel(rT, n_rel):
    def body(x_ref, o_ref):
        lo = lax.bitcast_convert_type(
            x_ref[0:16, :].astype(jnp.bfloat16), jnp.uint16).astype(jnp.int32)
        hi = lax.bitcast_convert_type(
            x_ref[16:32, :].astype(jnp.bfloat16), jnp.uint16).astype(jnp.int32)
        o_ref[...] = lo | (hi << 16)

    return pl.pallas_call(
        body,
        out_shape=jax.ShapeDtypeStruct((16, n_rel), jnp.int32),
    )(rT)


def _sc_stage(h, r, t, packed, rreT, rimT, n_rows, n_rel):
    info = plsc.get_sparse_core_info()
    nc, ns = info.num_cores, info.num_subcores
    nw = nc * ns
    rows_per_w = n_rows // nw
    n_chunks = rows_per_w // _CHUNK
    n_groups = _CHUNK // 16
    mesh = plsc.VectorSubcoreMesh(core_axis_name="c", subcore_axis_name="s")

    @functools.partial(
        pl.kernel,
        mesh=mesh,
        compiler_params=pltpu.CompilerParams(needs_layout_passes=False),
        out_type=(
            jax.ShapeDtypeStruct((n_rows,), jnp.float32),
            jax.ShapeDtypeStruct((nw, 16), jnp.float32),
        ),
        scratch_types=[
            pltpu.VMEM((2, _CHUNK), jnp.int32),
            pltpu.VMEM((2, _CHUNK), jnp.int32),
            pltpu.VMEM((2, _CHUNK), jnp.int32),
            pltpu.VMEM((2, 2 * _CHUNK), jnp.int32),
            pltpu.VMEM((2, 2 * _CHUNK, 128), jnp.int32),
            pltpu.VMEM((2 * _CHUNK, 256), jnp.float32),
            pltpu.VMEM((16, n_rel), jnp.int32),
            pltpu.VMEM((16, n_rel), jnp.int32),
            pltpu.VMEM((2, _CHUNK), jnp.float32),
            pltpu.VMEM((16,), jnp.float32),
            pltpu.SemaphoreType.DMA,
            pltpu.SemaphoreType.DMA,
            pltpu.SemaphoreType.DMA,
        ],
    )
    def sc_kernel(h_hbm, r_hbm, t_hbm, packed_hbm, rreT_hbm, rimT_hbm,
                  score_out, sq_out,
                  hi_v, ri_v, ti_v, q_v, bg, bf, vrr, vri,
                  sc_v, sq_v, sem_i, sem_g, sem_o):
        wid = lax.axis_index("s") * nc + lax.axis_index("c")
        base_w = wid * rows_per_w

        rd0 = pltpu.async_copy(rreT_hbm, vrr, sem_o)
        rd1 = pltpu.async_copy(rimT_hbm, vri, sem_o)
        sq_v[...] = jnp.zeros((16,), jnp.float32)

        def load_idx(g, b):
            base = base_w + g * _CHUNK
            return (
                pltpu.async_copy(h_hbm.at[pl.ds(base, _CHUNK)], hi_v.at[b], sem_i),
                pltpu.async_copy(r_hbm.at[pl.ds(base, _CHUNK)], ri_v.at[b], sem_i),
                pltpu.async_copy(t_hbm.at[pl.ds(base, _CHUNK)], ti_v.at[b], sem_i),
            )

        def start_gather(b):
            for g2 in range(n_groups):
                sl = pl.ds(g2 * 16, 16)
                e = hi_v[b, sl]
                q_v[b, sl] = lax.shift_left(lax.shift_right_logical(e, 9), 7) | (e & 127)
                e = ti_v[b, sl]
                q_v[b, pl.ds(_CHUNK + g2 * 16, 16)] = (
                    lax.shift_left(lax.shift_right_logical(e, 9), 7) | (e & 127))
            return pltpu.async_copy(packed_hbm.at[q_v.at[b]], bg.at[b], sem_g)

        def compute(g, b):
            def conv_row(j, carry):
                for gg in range(8):
                    v = bg[b, j, pl.ds(16 * gg, 16)]
                    lo = plsc.bitcast(v << 16, jnp.float32)
                    hi = plsc.bitcast(v & jnp.int32(-65536), jnp.float32)
                    bf[j, pl.ds(32 * gg, 16)] = lo
                    bf[j, pl.ds(32 * gg + 16, 16)] = hi
                return carry

            lax.fori_loop(0, 2 * _CHUNK, conv_row, 0)

            def group(g2, sq):
                sl = pl.ds(g2 * 16, 16)
                j16 = lax.iota(jnp.int32, 16) + g2 * 16
                offh = (lax.shift_right_logical(hi_v[b, sl], 7) & 3) * 64
                offt = (lax.shift_right_logical(ti_v[b, sl], 7) & 3) * 64
                r16 = ri_v[b, sl]
                score = jnp.zeros((16,), jnp.float32)
                jt16 = j16 + _CHUNK
                for c in range(_DIM):
                    cc = jnp.full((16,), c & 15, jnp.int32)
                    reh = plsc.load_gather(bf, [j16, offh + c])
                    imh = plsc.load_gather(bf, [j16, offh + (32 + c)])
                    ret_ = plsc.load_gather(bf, [jt16, offt + c])
                    imt = plsc.load_gather(bf, [jt16, offt + (32 + c)])
                    vr = plsc.load_gather(vrr, [cc, r16])
                    vi = plsc.load_gather(vri, [cc, r16])
                    if c < 16:
                        rre = plsc.bitcast(vr << 16, jnp.float32)
                        rim = plsc.bitcast(vi << 16, jnp.float32)
                    else:
                        rre = plsc.bitcast(vr & jnp.int32(-65536), jnp.float32)
                        rim = plsc.bitcast(vi & jnp.int32(-65536), jnp.float32)
                    score = score + (rre * (reh * ret_ + imh * imt)
                                     + rim * (reh * imt - imh * ret_))
                    sq = sq + (reh * reh + imh * imh + ret_ * ret_
                               + imt * imt + rre * rre + rim * rim)
                sc_v[b, sl] = score
                return sq

            sq = lax.fori_loop(0, n_groups, group, sq_v[...])
            sq_v[...] = sq
            base = base_w + g * _CHUNK
            return pltpu.async_copy(
                sc_v.at[b], score_out.at[pl.ds(base, _CHUNK)], sem_o)

        i0 = load_idx(0, 0)
        rd0.wait()
        rd1.wait()
        for d in i0:
            d.wait()
        g_prev = start_gather(0)
        out_prev = None
        for g in range(n_chunks):
            b = g & 1
            if g + 1 < n_chunks:
                i1 = load_idx(g + 1, b ^ 1)
            g_prev.wait()
            if g + 1 < n_chunks:
                for d in i1:
                    d.wait()
                g_next = start_gather(b ^ 1)
            if out_prev is not None:
                out_prev.wait()
            out_prev = compute(g, b)
            if g + 1 < n_chunks:
                g_prev = g_next
        out_prev.wait()

        pltpu.sync_copy(sq_v, sq_out.at[wid])

    return sc_kernel(h, r, t, packed, rreT, rimT)


def _tc_reduce(score, sq, n_rows):
    def body(s_ref, sq_ref, o_ref):
        s = s_ref[...]
        sp = jnp.maximum(s, 0.0) + jnp.log(1.0 + jnp.exp(-jnp.abs(s)))
        loss = jnp.sum(sp) * (1.0 / n_rows)
        regul = jnp.sum(sq_ref[...]) * (1.0 / (n_rows * _DIM))
        o_ref[0, 0] = loss + _LAMBDA * regul

    out = pl.pallas_call(
        body,
        out_shape=jax.ShapeDtypeStruct((1, 1), jnp.float32),
        out_specs=pl.BlockSpec(memory_space=pltpu.SMEM),
    )(score.reshape(n_rows // 128, 128), sq)
    return out[0, 0]


def kernel(pos_h, pos_r, pos_t, neg_h, neg_r, neg_t, ent_re, ent_im, rel_re, rel_im):
    h = jnp.concatenate([pos_h, neg_h])
    r = jnp.concatenate([pos_r, neg_r])
    t = jnp.concatenate([pos_t, neg_t])
    n_rows = h.shape[0]
    n_rel = rel_re.shape[0]
    packed = _pack_tables(ent_re.T, ent_im.T, ent_re.shape[0])
    score, sq = _sc_stage(h, r, t, packed,
                          _pack_rel(rel_re.T, n_rel),
                          _pack_rel(rel_im.T, n_rel),
                          n_rows, n_rel)
    return _tc_reduce(score, sq, n_rows)


# speedup vs baseline: 1.1433x; 1.1433x over previous
import functools

import jax
import jax.numpy as jnp
from jax import lax
from jax.experimental import pallas as pl
from jax.experimental.pallas import tpu as pltpu
from jax.experimental.pallas import tpu_sc as plsc

_DIM = 32
_LAMBDA = 0.01
_CHUNK = 64
_EB = 4096


def _pack_tables(ereT, eimT, n_ent):
    n_blocks = (n_ent + _EB - 1) // _EB
    n_sub = _EB // 256

    def body(a_ref, b_ref, o_ref):
        for k in range(n_sub):
            s0 = pl.ds(256 * k, 128)
            s1 = pl.ds(256 * k + 128, 128)
            stacked = jnp.concatenate(
                [a_ref[:, s0], b_ref[:, s0], a_ref[:, s1], b_ref[:, s1]],
                axis=0)
            o_ref[pl.ds(128 * k, 128), :] = stacked.T

    return pl.pallas_call(
        body,
        grid=(n_blocks,),
        in_specs=[
            pl.BlockSpec((_DIM, _EB), lambda g: (0, g)),
            pl.BlockSpec((_DIM, _EB), lambda g: (0, g)),
        ],
        out_specs=pl.BlockSpec((_EB // 2, 128), lambda g: (g, 0)),
        out_shape=jax.ShapeDtypeStruct((n_blocks * (_EB // 2), 128), jnp.float32),
    )(ereT, eimT)


def _sc_stage(h, r, t, packed, rreT, rimT, n_rows, n_rel):
    info = plsc.get_sparse_core_info()
    nc, ns = info.num_cores, info.num_subcores
    nw = nc * ns
    rows_per_w = n_rows // nw
    n_chunks = rows_per_w // _CHUNK
    n_groups = _CHUNK // 16
    mesh = plsc.VectorSubcoreMesh(core_axis_name="c", subcore_axis_name="s")

    @functools.partial(
        pl.kernel,
        mesh=mesh,
        compiler_params=pltpu.CompilerParams(needs_layout_passes=False),
        out_type=(
            jax.ShapeDtypeStruct((n_rows,), jnp.float32),
            jax.ShapeDtypeStruct((nw, 16), jnp.float32),
        ),
        scratch_types=[
            pltpu.VMEM((2, _CHUNK), jnp.int32),
            pltpu.VMEM((2, _CHUNK), jnp.int32),
            pltpu.VMEM((2, _CHUNK), jnp.int32),
            pltpu.VMEM((2, 2 * _CHUNK), jnp.int32),
            pltpu.VMEM((2, 2 * _CHUNK, 128), jnp.float32),
            pltpu.VMEM((_DIM, n_rel), jnp.float32),
            pltpu.VMEM((_DIM, n_rel), jnp.float32),
            pltpu.VMEM((2, _CHUNK), jnp.float32),
            pltpu.VMEM((16,), jnp.float32),
            pltpu.SemaphoreType.DMA,
            pltpu.SemaphoreType.DMA,
            pltpu.SemaphoreType.DMA,
        ],
    )
    def sc_kernel(h_hbm, r_hbm, t_hbm, packed_hbm, rreT_hbm, rimT_hbm,
                  score_out, sq_out,
                  hi_v, ri_v, ti_v, q_v, bg, vrr, vri,
                  sc_v, sq_v, sem_i, sem_g, sem_o):
        wid = lax.axis_index("s") * nc + lax.axis_index("c")
        base_w = wid * rows_per_w

        rd0 = pltpu.async_copy(rreT_hbm, vrr, sem_o)
        rd1 = pltpu.async_copy(rimT_hbm, vri, sem_o)
        sq_v[...] = jnp.zeros((16,), jnp.float32)

        def load_idx(g, b):
            base = pl.multiple_of(base_w + g * _CHUNK, _CHUNK)
            return (
                pltpu.async_copy(h_hbm.at[pl.ds(base, _CHUNK)], hi_v.at[b], sem_i),
                pltpu.async_copy(r_hbm.at[pl.ds(base, _CHUNK)], ri_v.at[b], sem_i),
                pltpu.async_copy(t_hbm.at[pl.ds(base, _CHUNK)], ti_v.at[b], sem_i),
            )

        def start_gather(b):
            for g2 in range(n_groups):
                sl = pl.ds(g2 * 16, 16)
                e = hi_v[b, sl]
                q_v[b, sl] = lax.shift_left(lax.shift_right_logical(e, 8), 7) | (e & 127)
                e = ti_v[b, sl]
                q_v[b, pl.ds(_CHUNK + g2 * 16, 16)] = (
                    lax.shift_left(lax.shift_right_logical(e, 8), 7) | (e & 127))
            return pltpu.async_copy(packed_hbm.at[q_v.at[b]], bg.at[b], sem_g)

        def compute(g, b):
            def group(g2, sq):
                sl = pl.ds(g2 * 16, 16)
                j16 = lax.iota(jnp.int32, 16) + g2 * 16
                offh = (lax.shift_right_logical(hi_v[b, sl], 7) & 1) * 64
                offt = (lax.shift_right_logical(ti_v[b, sl], 7) & 1) * 64
                r16 = ri_v[b, sl]
                score = jnp.zeros((16,), jnp.float32)
                jt16 = j16 + _CHUNK
                for c in range(_DIM):
                    cc = jnp.full((16,), c, jnp.int32)
                    reh = plsc.load_gather(bg.at[b], [j16, offh + c])
                    imh = plsc.load_gather(bg.at[b], [j16, offh + (32 + c)])
                    ret_ = plsc.load_gather(bg.at[b], [jt16, offt + c])
                    imt = plsc.load_gather(bg.at[b], [jt16, offt + (32 + c)])
                    rre = plsc.load_gather(vrr, [cc, r16])
                    rim = plsc.load_gather(vri, [cc, r16])
                    score = score + (rre * (reh * ret_ + imh * imt)
                                     + rim * (reh * imt - imh * ret_))
                    sq = sq + (reh * reh + imh * imh + ret_ * ret_
                               + imt * imt + rre * rre + rim * rim)
                sc_v[b, sl] = score
                return sq

            sq = lax.fori_loop(0, n_groups, group, sq_v[...])
            sq_v[...] = sq
            base = pl.multiple_of(base_w + g * _CHUNK, _CHUNK)
            return pltpu.async_copy(
                sc_v.at[b], score_out.at[pl.ds(base, _CHUNK)], sem_o)

        def wait_idx(b):
            pltpu.make_async_copy(
                h_hbm.at[pl.ds(0, _CHUNK)], hi_v.at[b], sem_i).wait()
            pltpu.make_async_copy(
                r_hbm.at[pl.ds(0, _CHUNK)], ri_v.at[b], sem_i).wait()
            pltpu.make_async_copy(
                t_hbm.at[pl.ds(0, _CHUNK)], ti_v.at[b], sem_i).wait()

        def wait_gather(b):
            pltpu.make_async_copy(
                packed_hbm.at[q_v.at[b]], bg.at[b], sem_g).wait()

        def wait_score(b):
            pltpu.make_async_copy(
                sc_v.at[b], score_out.at[pl.ds(0, _CHUNK)], sem_o).wait()

        load_idx(0, 0)
        rd0.wait()
        rd1.wait()
        wait_idx(0)
        start_gather(0)

        def step(g, carry):
            b = g & 1
            nb = b ^ 1
            more = g + 1 < n_chunks

            @pl.when(more)
            def _():
                load_idx(g + 1, nb)

            wait_gather(b)

            @pl.when(more)
            def _():
                wait_idx(nb)
                start_gather(nb)

            @pl.when(g > 0)
            def _():
                wait_score(nb)

            compute(g, b)
            return carry

        lax.fori_loop(0, n_chunks, step, 0)
        wait_score((n_chunks - 1) & 1)

        pltpu.sync_copy(sq_v, sq_out.at[wid])

    return sc_kernel(h, r, t, packed, rreT, rimT)


def _tc_reduce(score, sq, n_rows):
    def body(s_ref, sq_ref, o_ref):
        s = s_ref[...]
        sp = jnp.maximum(s, 0.0) + jnp.log(1.0 + jnp.exp(-jnp.abs(s)))
        loss = jnp.sum(sp) * (1.0 / n_rows)
        regul = jnp.sum(sq_ref[...]) * (1.0 / (n_rows * _DIM))
        o_ref[0, 0] = loss + _LAMBDA * regul

    out = pl.pallas_call(
        body,
        out_shape=jax.ShapeDtypeStruct((1, 1), jnp.float32),
        out_specs=pl.BlockSpec(memory_space=pltpu.SMEM),
    )(score.reshape(n_rows // 128, 128), sq)
    return out[0, 0]


def kernel(pos_h, pos_r, pos_t, neg_h, neg_r, neg_t, ent_re, ent_im, rel_re, rel_im):
    h = jnp.concatenate([pos_h, neg_h])
    r = jnp.concatenate([pos_r, neg_r])
    t = jnp.concatenate([pos_t, neg_t])
    n_rows = h.shape[0]
    packed = _pack_tables(ent_re.T, ent_im.T, ent_re.shape[0])
    score, sq = _sc_stage(h, r, t, packed, rel_re.T, rel_im.T,
                          n_rows, rel_re.shape[0])
    return _tc_reduce(score, sq, n_rows)
